# baseline (speedup 1.0000x reference)
import numpy as np
import jax
import jax.numpy as jnp
from jax.experimental import pallas as pl
from jax.experimental.pallas import tpu as pltpu

_GRAD_THRESHOLD = 0.5
_MIN_OPACITY = 0.05
_LOG2 = float(np.log(2.0))


def _pick_block(n, cap=1000):
    best = 8
    for b in range(8, cap + 1, 8):
        if n % b == 0:
            best = b
    return best


def _norm_body(sc_ref, out_ref):
    s = jnp.exp(sc_ref[...])
    out_ref[...] = jnp.sqrt(jnp.sum(s * s, axis=1, keepdims=True))


def _median_body(k1, k2, x_ref, thr_ref):
    x = x_ref[...]
    xi = jax.lax.bitcast_convert_type(x, jnp.int32)

    def cnt_le(t):
        return jnp.sum((xi <= t).astype(jnp.int32))

    def it(_, carry):
        lo, hi = carry
        mid = lo + (hi - lo) // 2
        pred = cnt_le(mid) >= k1
        lo2 = jnp.where(pred, lo, mid)
        hi2 = jnp.where(pred, mid, hi)
        return lo2, hi2

    lo0 = jnp.int32(-1)
    hi0 = jnp.int32(0x7F800000)
    _, a_int = jax.lax.fori_loop(0, 31, it, (lo0, hi0))
    neg_inf = jnp.float32(-np.inf)
    pos_inf = jnp.float32(np.inf)
    a = jnp.max(jnp.where(xi <= a_int, x, neg_inf))
    c_a = cnt_le(a_int)
    b = jnp.where(c_a >= k2, a, jnp.min(jnp.where(xi > a_int, x, pos_inf)))
    thr_ref[0, 0] = (a + b) * 0.5


def _main_body(thr_ref, pos_ref, sc_ref, rot_ref, op_ref, dc_ref, rest_ref,
               ga_ref, gc_ref, sn_ref, out_ref):
    thr = thr_ref[0, 0]
    cnts = jnp.maximum(gc_ref[...], 1).astype(jnp.float32)
    avg = ga_ref[...] / cnts
    gnorm = jnp.sqrt(jnp.sum(avg * avg, axis=1, keepdims=True))
    large = gnorm >= _GRAD_THRESHOLD
    pos = pos_ref[...]
    sc = sc_ref[...]
    asc = jnp.exp(sc)
    snorm = jnp.sqrt(jnp.sum(asc * asc, axis=1, keepdims=True))
    clone = large & (snorm <= thr)
    split = large & (snorm > thr)
    act_op = jax.nn.sigmoid(op_ref[...])
    keep = jnp.logical_not((act_op < _MIN_OPACITY) | split)

    rot = rot_ref[...]
    op = op_ref[...]
    dc = dc_ref[...]
    rest = rest_ref[...]

    def rowcat(p, s):
        return jnp.concatenate([p, s, rot, op, dc, rest], axis=1)

    base = rowcat(pos, sc)
    out_ref[0] = jnp.where(keep, base, 0.0)
    out_ref[1] = jnp.where(clone, base, 0.0)
    sp_sc = sc - _LOG2
    for i in range(2):
        spi = rowcat(pos + sn_ref[i] * asc, sp_sc)
        out_ref[2 + i] = jnp.where(split, spi, 0.0)


def _build(n, interpret=False):
    f32 = jnp.float32
    ba = _pick_block(n, 4000)
    na = n // ba
    norms_call = pl.pallas_call(
        _norm_body,
        grid=(na,),
        in_specs=[pl.BlockSpec((ba, 3), lambda i: (i, 0))],
        out_specs=pl.BlockSpec((ba, 1), lambda i: (i, 0)),
        out_shape=jax.ShapeDtypeStruct((n, 1), f32),
        interpret=interpret,
    )

    k1 = n // 2
    k2 = n // 2 + 1
    n8 = n // 8
    median_call = pl.pallas_call(
        lambda x_ref, t_ref: _median_body(k1, k2, x_ref, t_ref),
        in_specs=[pl.BlockSpec(memory_space=pltpu.VMEM)],
        out_specs=pl.BlockSpec(memory_space=pltpu.SMEM),
        out_shape=jax.ShapeDtypeStruct((1, 1), f32),
        interpret=interpret,
    )

    b = _pick_block(n, 1000)
    nb = n // b
    main_call = pl.pallas_call(
        _main_body,
        grid=(nb,),
        in_specs=[
            pl.BlockSpec(memory_space=pltpu.SMEM),
            pl.BlockSpec((b, 3), lambda i: (i, 0)),
            pl.BlockSpec((b, 3), lambda i: (i, 0)),
            pl.BlockSpec((b, 4), lambda i: (i, 0)),
            pl.BlockSpec((b, 1), lambda i: (i, 0)),
            pl.BlockSpec((b, 3), lambda i: (i, 0)),
            pl.BlockSpec((b, 9), lambda i: (i, 0)),
            pl.BlockSpec((b, 2), lambda i: (i, 0)),
            pl.BlockSpec((b, 1), lambda i: (i, 0)),
            pl.BlockSpec((2, b, 3), lambda i: (0, i, 0)),
        ],
        out_specs=pl.BlockSpec((4, b, 23), lambda i: (0, i, 0)),
        out_shape=jax.ShapeDtypeStruct((4, n, 23), f32),
        interpret=interpret,
    )

    def run(positions, scales, rotations, opacities, sh_dc, sh_rest,
            grad_accum, grad_count, split_noise):
        norms = norms_call(scales)
        thr = median_call(norms.reshape(8, n8))
        out4 = main_call(thr, positions, scales, rotations, opacities,
                         sh_dc, sh_rest, grad_accum,
                         grad_count.reshape(n, 1), split_noise)
        return out4.reshape(4 * n, 23)

    return run


_CACHE = {}


def kernel(positions, scales, rotations, opacities, sh_dc, sh_rest,
           grad_accum, grad_count, split_noise):
    n = positions.shape[0]
    if n not in _CACHE:
        _CACHE[n] = _build(n)
    return _CACHE[n](positions, scales, rotations, opacities, sh_dc, sh_rest,
                     grad_accum, grad_count, split_noise)
